# baseline (speedup 1.0000x reference)
import functools

import jax
import jax.numpy as jnp
from jax import lax
from jax.experimental import pallas as pl
from jax.experimental.pallas import tpu as pltpu
from jax.experimental.pallas import tpu_sc as plsc

HD = 64
INTD = 21
L = 16
NC, NS = 2, 16
NW = NC * NS
C = 128


def _proj_small_body(e_int, e_test, e_tag, w_int, w_test, w_tag, b,
                     p_int, p_test, p_tag):
    p_int[...] = jnp.dot(e_int[...], w_int[...],
                         preferred_element_type=jnp.float32) + b[...]
    p_test[...] = jnp.dot(e_test[...], w_test[...],
                          preferred_element_type=jnp.float32)
    p_tag[...] = jnp.dot(e_tag[...], w_tag[...],
                         preferred_element_type=jnp.float32)


def _proj_q_body(e_q, w_q, p_q):
    p_q[...] = jnp.dot(e_q[...], w_q[...], preferred_element_type=jnp.float32)


S_ = 200
RPB = 16
IB = RPB * S_
CA, CB = 128, 72


def _gather_sum_body(n_rows_w, n_blocks, off_test, off_tag,
                     i_int, i_test, i_q, i_tag,
                     p_small, p_q,
                     out_hbm,
                     idx0, idx1, idx2, idx3,
                     gb, ob, ps_small,
                     sem_g0, sem_g1, sem_s0, sem_s1, sem_ob0, sem_ob1):
    sid = lax.axis_index("s")
    wid = sid * NC + lax.axis_index("c")
    wrow0 = wid * n_rows_w
    wbase = wrow0 * S_
    sem_g = (sem_g0, sem_g1)
    sem_s = (sem_s0, sem_s1)
    sem_ob = (sem_ob0, sem_ob1)

    n_small_chunks = p_small.shape[0] // C
    for rep in range((n_small_chunks + NS - 1) // NS):
        ck = sid + rep * NS

        @pl.when(ck < n_small_chunks)
        def _stage():
            pltpu.sync_copy(p_small.at[pl.ds(ck * C, C)], gb[0][0])
            pltpu.sync_copy(gb[0][0], ps_small.at[pl.ds(ck * C, C)])

    plsc.subcore_barrier()

    SUBS = ((0, CA, 2), (CA, CB, 1))

    def fire(j, slot):
        r, sub = j // 2, j % 2
        off0, cnt, nss = SUBS[sub]
        base = r * S_ + off0
        per = cnt // nss
        descs = []
        for t, (tab, idx) in enumerate(
                ((ps_small, idx0), (ps_small, idx1), (p_q, idx2), (ps_small, idx3))):
            sem = sem_g[slot] if t == 2 else sem_s[slot]
            for ss in range(nss):
                descs.append(pltpu.async_copy(
                    tab.at[idx.at[pl.ds(base + ss * per, per)]],
                    gb[t][slot].at[pl.ds(ss * per, per)],
                    sem))
        return descs

    def block_body(blk, carry):
        boff = wbase + blk * IB
        row0 = wrow0 + blk * RPB
        pltpu.sync_copy(i_int.at[pl.ds(boff, IB)], idx0)
        pltpu.sync_copy(i_test.at[pl.ds(boff, IB)], idx1)
        pltpu.sync_copy(i_q.at[pl.ds(boff, IB)], idx2)
        pltpu.sync_copy(i_tag.at[pl.ds(boff, IB)], idx3)

        @plsc.parallel_loop(0, IB // L, step=1, unroll=4)
        def _offset(k):
            s = pl.ds(k * L, L)
            idx1[s] = idx1[s] + off_test
            idx3[s] = idx3[s] + off_tag

        descs = fire(0, 0)
        ob_descs = [None, None]
        for j in range(2 * RPB):
            slot = j % 2
            r, sub = j // 2, j % 2
            descs_next = fire(j + 1, 1 - slot) if j + 1 < 2 * RPB else None
            for d in descs:
                d.wait()
            if ob_descs[slot] is not None:
                ob_descs[slot].wait()
            b0, b1, b2, b3 = (gb[t][slot] for t in range(4))
            obuf = ob[slot]
            off0, cnt, _ = SUBS[sub]

            @plsc.parallel_loop(0, cnt, step=1, unroll=2)
            def add_body(rr):
                for cg in range(HD // L):
                    s = pl.ds(cg * L, L)
                    obuf[rr, s] = b0[rr, s] + b1[rr, s] + b2[rr, s] + b3[rr, s]

            ob_descs[slot] = pltpu.async_copy(
                obuf, out_hbm.at[row0 + r, pl.ds(off0, cnt)], sem_ob[slot])
            descs = descs_next
        for d in ob_descs:
            if d is not None:
                d.wait()
        return carry

    lax.fori_loop(0, n_blocks, block_body, 0)


def kernel(test, question, tag, correct, mask, interaction,
           emb_interaction, emb_test, emb_question, emb_tag,
           W_comb, b_comb):
    B, S = interaction.shape
    N = B * S
    n_rows_w = B // NW
    n_blocks = n_rows_w // RPB

    w_int = W_comb[0 * INTD:1 * INTD]
    w_test = W_comb[1 * INTD:2 * INTD]
    w_q = W_comb[2 * INTD:3 * INTD]
    w_tag = W_comb[3 * INTD:4 * INTD]
    b2d = b_comb.reshape(1, HD)

    n_int = emb_interaction.shape[0]
    n_test = emb_test.shape[0]
    n_q = emb_question.shape[0]
    n_tag = emb_tag.shape[0]

    p_int, p_test, p_tag = pl.pallas_call(
        _proj_small_body,
        out_shape=[
            jax.ShapeDtypeStruct((n_int, HD), jnp.float32),
            jax.ShapeDtypeStruct((n_test, HD), jnp.float32),
            jax.ShapeDtypeStruct((n_tag, HD), jnp.float32),
        ],
    )(emb_interaction, emb_test, emb_tag, w_int, w_test, w_tag, b2d)

    RQ = 8192
    grid_q = (n_q + RQ - 1) // RQ
    p_q = pl.pallas_call(
        _proj_q_body,
        grid=(grid_q,),
        in_specs=[
            pl.BlockSpec((RQ, INTD), lambda i: (i, 0)),
            pl.BlockSpec((INTD, HD), lambda i: (0, 0)),
        ],
        out_specs=pl.BlockSpec((RQ, HD), lambda i: (i, 0)),
        out_shape=jax.ShapeDtypeStruct((n_q, HD), jnp.float32),
    )(emb_question, w_q)

    n_small = n_int + n_test + n_tag
    n_small_pad = ((n_small + C - 1) // C) * C
    p_small = jnp.concatenate(
        [p_int, p_test, p_tag,
         jnp.zeros((n_small_pad - n_small, HD), jnp.float32)], axis=0)
    i_int_f = interaction.reshape(N)
    i_test_f = test.reshape(N)
    i_tag_f = tag.reshape(N)

    mesh = plsc.VectorSubcoreMesh(core_axis_name="c", subcore_axis_name="s")
    sc = functools.partial(
        pl.kernel,
        out_type=jax.ShapeDtypeStruct((B, S, HD), jnp.float32),
        mesh=mesh,
        compiler_params=pltpu.CompilerParams(use_tc_tiling_on_sc=False),
        scratch_types=[
            pltpu.VMEM((IB,), jnp.int32),
            pltpu.VMEM((IB,), jnp.int32),
            pltpu.VMEM((IB,), jnp.int32),
            pltpu.VMEM((IB,), jnp.int32),
            [[pltpu.VMEM((C, HD), jnp.float32) for _ in range(2)]
             for _ in range(4)],
            [pltpu.VMEM((CA, HD), jnp.float32),
             pltpu.VMEM((CB, HD), jnp.float32)],
            pltpu.VMEM_SHARED((n_small_pad, HD), jnp.float32),
            pltpu.SemaphoreType.DMA,
            pltpu.SemaphoreType.DMA,
            pltpu.SemaphoreType.DMA,
            pltpu.SemaphoreType.DMA,
            pltpu.SemaphoreType.DMA,
            pltpu.SemaphoreType.DMA,
        ],
    )(functools.partial(_gather_sum_body, n_rows_w, n_blocks,
                        n_int, n_int + n_test))

    X = sc(i_int_f, i_test_f, question.reshape(N), i_tag_f,
           p_small, p_q)
    return (X, B)


# speedup vs baseline: 1.1068x; 1.1068x over previous
import functools

import jax
import jax.numpy as jnp
from jax import lax
from jax.experimental import pallas as pl
from jax.experimental.pallas import tpu as pltpu
from jax.experimental.pallas import tpu_sc as plsc

HD = 64
INTD = 21
L = 16
NC, NS = 2, 16
NW = NC * NS
C = 128


def _proj_small_body(e_int, e_test, e_tag, w_int, w_test, w_tag, b,
                     p_int, p_test, p_tag):
    p_int[...] = jnp.dot(e_int[...], w_int[...],
                         preferred_element_type=jnp.float32) + b[...]
    p_test[...] = jnp.dot(e_test[...], w_test[...],
                          preferred_element_type=jnp.float32)
    p_tag[...] = jnp.dot(e_tag[...], w_tag[...],
                         preferred_element_type=jnp.float32)


def _proj_q_body(e_q, w_q, p_q):
    p_q[...] = jnp.dot(e_q[...], w_q[...], preferred_element_type=jnp.float32)


IBC = 25
IB = IBC * C


def _gather_sum_body(n_per_w, n_blocks, off_test, off_tag,
                     i_int, i_test, i_q, i_tag,
                     p_small, p_q,
                     out_hbm,
                     idx0, idx1, idx2, idx3,
                     gb, ob, ps_small,
                     sem_g0, sem_g1, sem_s0, sem_s1, sem_ob0, sem_ob1):
    sid = lax.axis_index("s")
    wid = sid * NC + lax.axis_index("c")
    wbase = wid * n_per_w
    sem_g = (sem_g0, sem_g1)
    sem_s = (sem_s0, sem_s1)
    sem_ob = (sem_ob0, sem_ob1)

    n_small_chunks = p_small.shape[0] // C
    for rep in range((n_small_chunks + NS - 1) // NS):
        ck = sid + rep * NS

        @pl.when(ck < n_small_chunks)
        def _stage():
            pltpu.sync_copy(p_small.at[pl.ds(ck * C, C)], gb[0][0])
            pltpu.sync_copy(gb[0][0], ps_small.at[pl.ds(ck * C, C)])

    plsc.subcore_barrier()

    SS = 4
    SC_ = C // SS

    def fire(j, slot):
        descs = []
        for t, (tab, idx) in enumerate(
                ((ps_small, idx0), (ps_small, idx1), (p_q, idx2), (ps_small, idx3))):
            sem = sem_g[slot] if t == 2 else sem_s[slot]
            for ss in range(SS):
                off = j * C + ss * SC_
                descs.append(pltpu.async_copy(
                    tab.at[idx.at[pl.ds(off, SC_)]],
                    gb[t][slot].at[pl.ds(ss * SC_, SC_)],
                    sem))
        return descs

    def block_body(blk, carry):
        boff = wbase + blk * IB
        pltpu.sync_copy(i_int.at[pl.ds(boff, IB)], idx0)
        pltpu.sync_copy(i_test.at[pl.ds(boff, IB)], idx1)
        pltpu.sync_copy(i_q.at[pl.ds(boff, IB)], idx2)
        pltpu.sync_copy(i_tag.at[pl.ds(boff, IB)], idx3)

        @plsc.parallel_loop(0, IB // L, step=1, unroll=4)
        def _offset(k):
            s = pl.ds(k * L, L)
            idx1[s] = idx1[s] + off_test
            idx3[s] = idx3[s] + off_tag

        descs = fire(0, 0)
        ob_descs = [None, None]
        for j in range(IBC):
            slot = j % 2
            descs_next = fire(j + 1, 1 - slot) if j + 1 < IBC else None
            for d in descs:
                d.wait()
            if ob_descs[slot] is not None:
                ob_descs[slot].wait()
            b0, b1, b2, b3 = (gb[t][slot] for t in range(4))
            obuf = ob[slot]

            @plsc.parallel_loop(0, C // 2, step=1, unroll=1)
            def add_body(r2):
                for half in range(2):
                    t = 2 * r2 + half
                    for cg in range(HD // L):
                        s = pl.ds(cg * L, L)
                        so = pl.ds(half * HD + cg * L, L)
                        obuf[r2, so] = b0[t, s] + b1[t, s] + b2[t, s] + b3[t, s]

            base = boff + j * C
            ob_descs[slot] = pltpu.async_copy(
                obuf, out_hbm.at[pl.ds(base // 2, C // 2)], sem_ob[slot])
            descs = descs_next
        for d in ob_descs:
            if d is not None:
                d.wait()
        return carry

    lax.fori_loop(0, n_blocks, block_body, 0)


def kernel(test, question, tag, correct, mask, interaction,
           emb_interaction, emb_test, emb_question, emb_tag,
           W_comb, b_comb):
    B, S = interaction.shape
    N = B * S
    n_per_w = N // NW
    n_blocks = n_per_w // IB

    w_int = W_comb[0 * INTD:1 * INTD]
    w_test = W_comb[1 * INTD:2 * INTD]
    w_q = W_comb[2 * INTD:3 * INTD]
    w_tag = W_comb[3 * INTD:4 * INTD]
    b2d = b_comb.reshape(1, HD)

    n_int = emb_interaction.shape[0]
    n_test = emb_test.shape[0]
    n_q = emb_question.shape[0]
    n_tag = emb_tag.shape[0]

    p_int, p_test, p_tag = pl.pallas_call(
        _proj_small_body,
        out_shape=[
            jax.ShapeDtypeStruct((n_int, HD), jnp.float32),
            jax.ShapeDtypeStruct((n_test, HD), jnp.float32),
            jax.ShapeDtypeStruct((n_tag, HD), jnp.float32),
        ],
    )(emb_interaction, emb_test, emb_tag, w_int, w_test, w_tag, b2d)

    RQ = 8192
    grid_q = (n_q + RQ - 1) // RQ
    p_q = pl.pallas_call(
        _proj_q_body,
        grid=(grid_q,),
        in_specs=[
            pl.BlockSpec((RQ, INTD), lambda i: (i, 0)),
            pl.BlockSpec((INTD, HD), lambda i: (0, 0)),
        ],
        out_specs=pl.BlockSpec((RQ, HD), lambda i: (i, 0)),
        out_shape=jax.ShapeDtypeStruct((n_q, HD), jnp.float32),
    )(emb_question, w_q)

    n_small = n_int + n_test + n_tag
    n_small_pad = ((n_small + C - 1) // C) * C
    p_small = jnp.concatenate(
        [p_int, p_test, p_tag,
         jnp.zeros((n_small_pad - n_small, HD), jnp.float32)], axis=0)
    i_int_f = interaction.reshape(N)
    i_test_f = test.reshape(N)
    i_tag_f = tag.reshape(N)

    mesh = plsc.VectorSubcoreMesh(core_axis_name="c", subcore_axis_name="s")
    sc = functools.partial(
        pl.kernel,
        out_type=jax.ShapeDtypeStruct((N // 2, 2 * HD), jnp.float32),
        mesh=mesh,
        compiler_params=pltpu.CompilerParams(use_tc_tiling_on_sc=False),
        scratch_types=[
            pltpu.VMEM((IB,), jnp.int32),
            pltpu.VMEM((IB,), jnp.int32),
            pltpu.VMEM((IB,), jnp.int32),
            pltpu.VMEM((IB,), jnp.int32),
            [[pltpu.VMEM((C, HD), jnp.float32) for _ in range(2)]
             for _ in range(4)],
            [pltpu.VMEM((C // 2, 2 * HD), jnp.float32) for _ in range(2)],
            pltpu.VMEM_SHARED((n_small_pad, HD), jnp.float32),
            pltpu.SemaphoreType.DMA,
            pltpu.SemaphoreType.DMA,
            pltpu.SemaphoreType.DMA,
            pltpu.SemaphoreType.DMA,
            pltpu.SemaphoreType.DMA,
            pltpu.SemaphoreType.DMA,
        ],
    )(functools.partial(_gather_sum_body, n_per_w, n_blocks,
                        n_int, n_int + n_test))

    out_pairs = sc(i_int_f, i_test_f, question.reshape(N), i_tag_f,
                   p_small, p_q)
    X = out_pairs.reshape(B, S, HD)
    return (X, B)
